# baseline (speedup 1.0000x reference)
import functools

import jax
import jax.numpy as jnp
from jax import lax
from jax.experimental import pallas as pl
from jax.experimental.pallas import tpu as pltpu
from jax.experimental.pallas import tpu_sc as plsc

QN = 1024
KN = 16384
DN = 256
CN = 1000
CP = 1024
TOP = 5
KB = 512
NB = KN // KB

_INF = float("inf")
_BIGI = 2**30


def _topk_body(x_ref, a_ref, w_ref, ix_ref, vals, idxs):
    k = pl.program_id(0)

    @pl.when(k == 0)
    def _init():
        vals[...] = jnp.full((16, QN), _INF, jnp.float32)
        idxs[...] = jnp.zeros((16, QN), jnp.int32)

    a = a_ref[...]
    x = x_ref[...]
    a2 = jnp.sum(a * a, axis=1, keepdims=True)
    dot = lax.dot_general(a, x, (((1,), (1,)), ((), ())),
                          preferred_element_type=jnp.float32)
    s = a2 + dot

    riota = lax.broadcasted_iota(jnp.int32, (KB, QN), 0)
    for j in range(TOP):
        m = jnp.min(s, axis=0, keepdims=True)
        p = jnp.min(jnp.where(s == m, riota, _BIGI), axis=0, keepdims=True)
        vals[pl.ds(TOP + j, 1), :] = m
        idxs[pl.ds(TOP + j, 1), :] = p + k * KB
        s = jnp.where(riota == p, _INF, s)

    v = vals[...]
    ix = idxs[...]
    r16 = lax.broadcasted_iota(jnp.int32, (16, QN), 0)
    for j in range(TOP):
        m = jnp.min(v, axis=0, keepdims=True)
        p = jnp.min(jnp.where(v == m, r16, _BIGI), axis=0, keepdims=True)
        sel = r16 == p
        vals[pl.ds(j, 1), :] = m
        idxs[pl.ds(j, 1), :] = jnp.min(jnp.where(sel, ix, _BIGI), axis=0, keepdims=True)
        v = jnp.where(sel, _INF, v)

    @pl.when(k == NB - 1)
    def _fin():
        b2 = 0.25 * jnp.sum(x * x, axis=1)[None, :]
        vv = vals[pl.ds(0, 8), :]
        d2 = jnp.maximum(vv + b2, 0.0)
        w_ref[...] = 1.0 / jnp.sqrt(d2)
        ix_ref[...] = idxs[pl.ds(0, 8), :]


def _topk_stage(X, clusters):
    return pl.pallas_call(
        _topk_body,
        grid=(NB,),
        in_specs=[
            pl.BlockSpec((QN, DN), lambda k: (0, 0)),
            pl.BlockSpec((KB, DN), lambda k: (k, 0)),
        ],
        out_specs=[
            pl.BlockSpec((8, QN), lambda k: (0, 0)),
            pl.BlockSpec((8, QN), lambda k: (0, 0)),
        ],
        out_shape=[
            jax.ShapeDtypeStruct((8, QN), jnp.float32),
            jax.ShapeDtypeStruct((8, QN), jnp.int32),
        ],
        scratch_shapes=[
            pltpu.VMEM((16, QN), jnp.float32),
            pltpu.VMEM((16, QN), jnp.int32),
        ],
    )(X, clusters)


_NC = 2
_NS = 16
_NW = _NC * _NS
_QPW = QN // _NW
_NG = QN // 16


def _scatter_sc_body(w_hbm, ix_hbm, labels_hbm, counts_hbm, out_hbm,
                     labels_v, counts_v, w_v, i_v, rows_v):
    wid = lax.axis_index("s") * _NC + lax.axis_index("c")
    pltpu.sync_copy(labels_hbm, labels_v)
    pltpu.sync_copy(counts_hbm, counts_v.at[pl.ds(0, CN)])
    lane = lax.iota(jnp.int32, 16)
    for g in range(_QPW // 16):
        gidx = wid * (_QPW // 16) + g
        qb = gidx * 16
        pltpu.sync_copy(w_hbm.at[gidx], w_v)
        pltpu.sync_copy(ix_hbm.at[gidx], i_v)

        def _base(c, carry):
            cv = counts_v[pl.ds(c * 16, 16)]
            bv = jnp.where(cv == 0.0, jnp.float32(-1.0), jnp.float32(0.0))
            for r in range(16):
                rows_v[r, pl.ds(c * 16, 16)] = bv
            return carry

        lax.fori_loop(0, CP // 16, _base, 0)
        for j in range(TOP):
            lbl = plsc.load_gather(labels_v, [i_v[j, :]])
            cnt = plsc.load_gather(counts_v, [lbl])
            plsc.addupdate_scatter(rows_v, [lane, lbl], w_v[j, :] * cnt)
        pltpu.sync_copy(rows_v, out_hbm.at[pl.ds(qb, 16), :])


@functools.cache
def _scatter_sc():
    return functools.partial(
        pl.kernel,
        mesh=plsc.VectorSubcoreMesh(core_axis_name="c", subcore_axis_name="s"),
        out_type=jax.ShapeDtypeStruct((QN, CP), jnp.float32),
        scratch_types=[
            pltpu.VMEM((KN,), jnp.int32),
            pltpu.VMEM((CP,), jnp.float32),
            pltpu.VMEM((8, 16), jnp.float32),
            pltpu.VMEM((8, 16), jnp.int32),
            pltpu.VMEM((16, CP), jnp.float32),
        ],
        compiler_params=pltpu.CompilerParams(needs_layout_passes=False),
    )(_scatter_sc_body)


def kernel(X, clusters, counts, labels):
    wT, ixT = _topk_stage(-2.0 * X, clusters)
    w_grp = wT.reshape(8, _NG, 16).transpose(1, 0, 2)
    ix_grp = ixT.reshape(8, _NG, 16).transpose(1, 0, 2)
    scores_p = _scatter_sc()(w_grp, ix_grp, labels, counts)
    return scores_p[:, :CN]


# speedup vs baseline: 1.0008x; 1.0008x over previous
import functools

import jax
import jax.numpy as jnp
from jax import lax
from jax.experimental import pallas as pl
from jax.experimental.pallas import tpu as pltpu
from jax.experimental.pallas import tpu_sc as plsc

QN = 1024
KN = 16384
DN = 256
CN = 1000
CP = 1024
TOP = 5
KB = 512
NB = KN // KB

_INF = float("inf")
_BIGI = 2**30


def _topk_body(x_ref, a_ref, w_ref, ix_ref, vals, idxs):
    k = pl.program_id(0)

    @pl.when(k == 0)
    def _init():
        vals[...] = jnp.full((16, QN), _INF, jnp.float32)
        idxs[...] = jnp.zeros((16, QN), jnp.int32)

    a = a_ref[...]
    x = x_ref[...]
    a2 = jnp.sum(a * a, axis=1, keepdims=True)
    dot = lax.dot_general(a, x, (((1,), (1,)), ((), ())),
                          preferred_element_type=jnp.float32)
    s = a2 + dot

    riota = lax.broadcasted_iota(jnp.int32, (KB, QN), 0)
    for j in range(TOP):
        m = jnp.min(s, axis=0, keepdims=True)
        p = jnp.min(jnp.where(s == m, riota, _BIGI), axis=0, keepdims=True)
        vals[pl.ds(TOP + j, 1), :] = m
        idxs[pl.ds(TOP + j, 1), :] = p + k * KB
        s = jnp.where(riota == p, _INF, s)

    v = vals[...]
    ix = idxs[...]
    r16 = lax.broadcasted_iota(jnp.int32, (16, QN), 0)
    nv, ni = [], []
    for j in range(TOP):
        m = jnp.min(v, axis=0, keepdims=True)
        p = jnp.min(jnp.where(v == m, r16, _BIGI), axis=0, keepdims=True)
        sel = r16 == p
        ni.append(jnp.min(jnp.where(sel, ix, _BIGI), axis=0, keepdims=True))
        nv.append(m)
        v = jnp.where(sel, _INF, v)
    vals[...] = jnp.concatenate(nv + [jnp.full((16 - TOP, QN), _INF, jnp.float32)], axis=0)
    idxs[...] = jnp.concatenate(ni + [jnp.zeros((16 - TOP, QN), jnp.int32)], axis=0)

    @pl.when(k == NB - 1)
    def _fin():
        b2 = 0.25 * jnp.sum(x * x, axis=1)[None, :]
        vv = vals[pl.ds(0, 8), :]
        d2 = jnp.maximum(vv + b2, 0.0)
        w_ref[...] = 1.0 / jnp.sqrt(d2)
        ix_ref[...] = idxs[pl.ds(0, 8), :]


def _topk_stage(X, clusters):
    return pl.pallas_call(
        _topk_body,
        grid=(NB,),
        in_specs=[
            pl.BlockSpec((QN, DN), lambda k: (0, 0)),
            pl.BlockSpec((KB, DN), lambda k: (k, 0)),
        ],
        out_specs=[
            pl.BlockSpec((8, QN), lambda k: (0, 0)),
            pl.BlockSpec((8, QN), lambda k: (0, 0)),
        ],
        out_shape=[
            jax.ShapeDtypeStruct((8, QN), jnp.float32),
            jax.ShapeDtypeStruct((8, QN), jnp.int32),
        ],
        scratch_shapes=[
            pltpu.VMEM((16, QN), jnp.float32),
            pltpu.VMEM((16, QN), jnp.int32),
        ],
    )(X, clusters)


_NC = 2
_NS = 16
_NW = _NC * _NS
_QPW = QN // _NW
_NG = QN // 16


def _scatter_sc_body(w_hbm, ix_hbm, labels_hbm, counts_hbm, out_hbm,
                     labels_v, counts_v, w_v, i_v, rows_v):
    wid = lax.axis_index("s") * _NC + lax.axis_index("c")
    pltpu.sync_copy(labels_hbm, labels_v)
    pltpu.sync_copy(counts_hbm, counts_v.at[pl.ds(0, CN)])
    lane = lax.iota(jnp.int32, 16)
    for g in range(_QPW // 16):
        gidx = wid * (_QPW // 16) + g
        qb = gidx * 16
        pltpu.sync_copy(w_hbm.at[gidx], w_v)
        pltpu.sync_copy(ix_hbm.at[gidx], i_v)

        def _base(c, carry):
            cv = counts_v[pl.ds(c * 16, 16)]
            bv = jnp.where(cv == 0.0, jnp.float32(-1.0), jnp.float32(0.0))
            for r in range(16):
                rows_v[r, pl.ds(c * 16, 16)] = bv
            return carry

        lax.fori_loop(0, CP // 16, _base, 0)
        for j in range(TOP):
            lbl = plsc.load_gather(labels_v, [i_v[j, :]])
            cnt = plsc.load_gather(counts_v, [lbl])
            plsc.addupdate_scatter(rows_v, [lane, lbl], w_v[j, :] * cnt)
        pltpu.sync_copy(rows_v, out_hbm.at[pl.ds(qb, 16), :])


@functools.cache
def _scatter_sc():
    return functools.partial(
        pl.kernel,
        mesh=plsc.VectorSubcoreMesh(core_axis_name="c", subcore_axis_name="s"),
        out_type=jax.ShapeDtypeStruct((QN, CP), jnp.float32),
        scratch_types=[
            pltpu.VMEM((KN,), jnp.int32),
            pltpu.VMEM((CP,), jnp.float32),
            pltpu.VMEM((8, 16), jnp.float32),
            pltpu.VMEM((8, 16), jnp.int32),
            pltpu.VMEM((16, CP), jnp.float32),
        ],
        compiler_params=pltpu.CompilerParams(needs_layout_passes=False),
    )(_scatter_sc_body)


def kernel(X, clusters, counts, labels):
    wT, ixT = _topk_stage(-2.0 * X, clusters)
    w_grp = wT.reshape(8, _NG, 16).transpose(1, 0, 2)
    ix_grp = ixT.reshape(8, _NG, 16).transpose(1, 0, 2)
    scores_p = _scatter_sc()(w_grp, ix_grp, labels, counts)
    return scores_p[:, :CN]


# speedup vs baseline: 1.0853x; 1.0844x over previous
import functools

import jax
import jax.numpy as jnp
from jax import lax
from jax.experimental import pallas as pl
from jax.experimental.pallas import tpu as pltpu
from jax.experimental.pallas import tpu_sc as plsc

QN = 1024
KN = 16384
DN = 256
CN = 1000
CP = 1024
TOP = 5
KB = 512
NB = KN // KB

_INF = float("inf")
_BIGI = 2**30


def _topk_body(x_ref, a_ref, w_ref, ix_ref, vals, idxs):
    k = pl.program_id(0)

    @pl.when(k == 0)
    def _init():
        vals[...] = jnp.full((16, QN), _INF, jnp.float32)
        idxs[...] = jnp.zeros((16, QN), jnp.int32)

    a = a_ref[...]
    x = x_ref[...]
    a2 = jnp.sum(a * a, axis=1, keepdims=True)
    dot = lax.dot_general(a, x, (((1,), (1,)), ((), ())),
                          preferred_element_type=jnp.float32)
    s = a2 - 2.0 * dot

    riota = lax.broadcasted_iota(jnp.int32, (KB, QN), 0)
    for j in range(TOP):
        m = jnp.min(s, axis=0, keepdims=True)
        p = jnp.min(jnp.where(s == m, riota, _BIGI), axis=0, keepdims=True)
        vals[pl.ds(TOP + j, 1), :] = m
        idxs[pl.ds(TOP + j, 1), :] = p + k * KB
        s = jnp.where(riota == p, _INF, s)

    v = vals[...]
    ix = idxs[...]
    r16 = lax.broadcasted_iota(jnp.int32, (16, QN), 0)
    nv, ni = [], []
    for j in range(TOP):
        m = jnp.min(v, axis=0, keepdims=True)
        p = jnp.min(jnp.where(v == m, r16, _BIGI), axis=0, keepdims=True)
        sel = r16 == p
        ni.append(jnp.min(jnp.where(sel, ix, _BIGI), axis=0, keepdims=True))
        nv.append(m)
        v = jnp.where(sel, _INF, v)
    vals[...] = jnp.concatenate(nv + [jnp.full((16 - TOP, QN), _INF, jnp.float32)], axis=0)
    idxs[...] = jnp.concatenate(ni + [jnp.zeros((16 - TOP, QN), jnp.int32)], axis=0)

    @pl.when(k == NB - 1)
    def _fin():
        b2 = jnp.sum(x * x, axis=1)[None, :]
        vv = vals[pl.ds(0, 8), :]
        d2 = jnp.maximum(vv + b2, 0.0)
        w_ref[...] = 1.0 / jnp.sqrt(d2)
        ix_ref[...] = idxs[pl.ds(0, 8), :]


def _topk_stage(X, clusters):
    return pl.pallas_call(
        _topk_body,
        grid=(NB,),
        in_specs=[
            pl.BlockSpec((QN, DN), lambda k: (0, 0)),
            pl.BlockSpec((KB, DN), lambda k: (k, 0)),
        ],
        out_specs=[
            pl.BlockSpec((8, QN), lambda k: (0, 0)),
            pl.BlockSpec((8, QN), lambda k: (0, 0)),
        ],
        out_shape=[
            jax.ShapeDtypeStruct((8, QN), jnp.float32),
            jax.ShapeDtypeStruct((8, QN), jnp.int32),
        ],
        scratch_shapes=[
            pltpu.VMEM((16, QN), jnp.float32),
            pltpu.VMEM((16, QN), jnp.int32),
        ],
    )(X, clusters)


_NC = 2
_NS = 16
_NW = _NC * _NS
_QPW = QN // _NW
_NG = QN // 16


def _scatter_sc_body(w_hbm, ix_hbm, labels_hbm, counts_hbm, out_hbm,
                     labels_v, counts_v, w_v, i_v, rows_v):
    wid = lax.axis_index("s") * _NC + lax.axis_index("c")
    pltpu.sync_copy(labels_hbm, labels_v)
    pltpu.sync_copy(counts_hbm, counts_v.at[pl.ds(0, CN)])
    lane = lax.iota(jnp.int32, 16)
    for g in range(_QPW // 16):
        gidx = wid * (_QPW // 16) + g
        qb = gidx * 16
        pltpu.sync_copy(w_hbm.at[gidx], w_v)
        pltpu.sync_copy(ix_hbm.at[gidx], i_v)

        def _base(c, carry):
            cv = counts_v[pl.ds(c * 16, 16)]
            bv = jnp.where(cv == 0.0, jnp.float32(-1.0), jnp.float32(0.0))
            for r in range(16):
                rows_v[r, pl.ds(c * 16, 16)] = bv
            return carry

        lax.fori_loop(0, CP // 16, _base, 0)
        for j in range(TOP):
            lbl = plsc.load_gather(labels_v, [i_v[j, :]])
            cnt = plsc.load_gather(counts_v, [lbl])
            plsc.addupdate_scatter(rows_v, [lane, lbl], w_v[j, :] * cnt)
        pltpu.sync_copy(rows_v, out_hbm.at[pl.ds(qb, 16), :])


@functools.cache
def _scatter_sc():
    return functools.partial(
        pl.kernel,
        mesh=plsc.VectorSubcoreMesh(core_axis_name="c", subcore_axis_name="s"),
        out_type=jax.ShapeDtypeStruct((QN, CP), jnp.float32),
        scratch_types=[
            pltpu.VMEM((KN,), jnp.int32),
            pltpu.VMEM((CP,), jnp.float32),
            pltpu.VMEM((8, 16), jnp.float32),
            pltpu.VMEM((8, 16), jnp.int32),
            pltpu.VMEM((16, CP), jnp.float32),
        ],
        compiler_params=pltpu.CompilerParams(needs_layout_passes=False),
    )(_scatter_sc_body)


def kernel(X, clusters, counts, labels):
    wT, ixT = _topk_stage(X, clusters)
    w_grp = wT.reshape(8, _NG, 16).transpose(1, 0, 2)
    ix_grp = ixT.reshape(8, _NG, 16).transpose(1, 0, 2)
    scores_p = _scatter_sc()(w_grp, ix_grp, labels, counts)
    return scores_p[:, :CN]


# speedup vs baseline: 1.1555x; 1.0647x over previous
import functools

import jax
import jax.numpy as jnp
from jax import lax
from jax.experimental import pallas as pl
from jax.experimental.pallas import tpu as pltpu
from jax.experimental.pallas import tpu_sc as plsc

QN = 1024
KN = 16384
DN = 256
CN = 1000
CP = 1024
TOP = 5
KB = 1024
NB = KN // KB

_INF = float("inf")
_BIGI = 2**30


def _topk_body(x_ref, a_ref, w_ref, ix_ref, vals, idxs):
    k = pl.program_id(0)

    @pl.when(k == 0)
    def _init():
        vals[...] = jnp.full((16, QN), _INF, jnp.float32)
        idxs[...] = jnp.zeros((16, QN), jnp.int32)

    a = a_ref[...]
    x = x_ref[...]
    a2 = jnp.sum(a * a, axis=1, keepdims=True)
    dot = lax.dot_general(a, x, (((1,), (1,)), ((), ())),
                          preferred_element_type=jnp.float32)
    s = a2 - 2.0 * dot

    riota = lax.broadcasted_iota(jnp.int32, (KB, QN), 0)
    for j in range(TOP):
        m = jnp.min(s, axis=0, keepdims=True)
        p = jnp.min(jnp.where(s == m, riota, _BIGI), axis=0, keepdims=True)
        vals[pl.ds(TOP + j, 1), :] = m
        idxs[pl.ds(TOP + j, 1), :] = p + k * KB
        s = jnp.where(riota == p, _INF, s)

    v = vals[...]
    ix = idxs[...]
    r16 = lax.broadcasted_iota(jnp.int32, (16, QN), 0)
    nv, ni = [], []
    for j in range(TOP):
        m = jnp.min(v, axis=0, keepdims=True)
        p = jnp.min(jnp.where(v == m, r16, _BIGI), axis=0, keepdims=True)
        sel = r16 == p
        ni.append(jnp.min(jnp.where(sel, ix, _BIGI), axis=0, keepdims=True))
        nv.append(m)
        v = jnp.where(sel, _INF, v)
    vals[...] = jnp.concatenate(nv + [jnp.full((16 - TOP, QN), _INF, jnp.float32)], axis=0)
    idxs[...] = jnp.concatenate(ni + [jnp.zeros((16 - TOP, QN), jnp.int32)], axis=0)

    @pl.when(k == NB - 1)
    def _fin():
        b2 = jnp.sum(x * x, axis=1)[None, :]
        vv = vals[pl.ds(0, 8), :]
        d2 = jnp.maximum(vv + b2, 0.0)
        w_ref[...] = 1.0 / jnp.sqrt(d2)
        ix_ref[...] = idxs[pl.ds(0, 8), :]


def _topk_stage(X, clusters):
    return pl.pallas_call(
        _topk_body,
        grid=(NB,),
        in_specs=[
            pl.BlockSpec((QN, DN), lambda k: (0, 0)),
            pl.BlockSpec((KB, DN), lambda k: (k, 0)),
        ],
        out_specs=[
            pl.BlockSpec((8, QN), lambda k: (0, 0)),
            pl.BlockSpec((8, QN), lambda k: (0, 0)),
        ],
        out_shape=[
            jax.ShapeDtypeStruct((8, QN), jnp.float32),
            jax.ShapeDtypeStruct((8, QN), jnp.int32),
        ],
        scratch_shapes=[
            pltpu.VMEM((16, QN), jnp.float32),
            pltpu.VMEM((16, QN), jnp.int32),
        ],
    )(X, clusters)


_NC = 2
_NS = 16
_NW = _NC * _NS
_QPW = QN // _NW
_NG = QN // 16


def _scatter_sc_body(w_hbm, ix_hbm, labels_hbm, counts_hbm, out_hbm,
                     labels_v, counts_v, w_v, i_v, rows_v):
    wid = lax.axis_index("s") * _NC + lax.axis_index("c")
    pltpu.sync_copy(labels_hbm, labels_v)
    pltpu.sync_copy(counts_hbm, counts_v.at[pl.ds(0, CN)])
    lane = lax.iota(jnp.int32, 16)
    for g in range(_QPW // 16):
        gidx = wid * (_QPW // 16) + g
        qb = gidx * 16
        pltpu.sync_copy(w_hbm.at[gidx], w_v)
        pltpu.sync_copy(ix_hbm.at[gidx], i_v)

        def _base(c, carry):
            off = jnp.minimum(c * 16, CN - 16)
            cv = counts_v[pl.ds(off, 16)]
            bv = jnp.where(cv == 0.0, jnp.float32(-1.0), jnp.float32(0.0))
            for r in range(16):
                rows_v[r, pl.ds(off, 16)] = bv
            return carry

        lax.fori_loop(0, (CN + 15) // 16, _base, 0)
        for j in range(TOP):
            lbl = plsc.load_gather(labels_v, [i_v[j, :]])
            cnt = plsc.load_gather(counts_v, [lbl])
            plsc.addupdate_scatter(rows_v, [lane, lbl], w_v[j, :] * cnt)
        pltpu.sync_copy(rows_v, out_hbm.at[pl.ds(qb, 16), :])


@functools.cache
def _scatter_sc():
    return functools.partial(
        pl.kernel,
        mesh=plsc.VectorSubcoreMesh(core_axis_name="c", subcore_axis_name="s"),
        out_type=jax.ShapeDtypeStruct((QN, CN), jnp.float32),
        scratch_types=[
            pltpu.VMEM((KN,), jnp.int32),
            pltpu.VMEM((CP,), jnp.float32),
            pltpu.VMEM((8, 16), jnp.float32),
            pltpu.VMEM((8, 16), jnp.int32),
            pltpu.VMEM((16, CN), jnp.float32),
        ],
        compiler_params=pltpu.CompilerParams(needs_layout_passes=False),
    )(_scatter_sc_body)


def kernel(X, clusters, counts, labels):
    wT, ixT = _topk_stage(X, clusters)
    w_grp = wT.reshape(8, _NG, 16).transpose(1, 0, 2)
    ix_grp = ixT.reshape(8, _NG, 16).transpose(1, 0, 2)
    return _scatter_sc()(w_grp, ix_grp, labels, counts)


# speedup vs baseline: 1.1658x; 1.0089x over previous
import functools

import jax
import jax.numpy as jnp
from jax import lax
from jax.experimental import pallas as pl
from jax.experimental.pallas import tpu as pltpu
from jax.experimental.pallas import tpu_sc as plsc

QN = 1024
KN = 16384
DN = 256
CN = 1000
CP = 1024
TOP = 5
KB = 1024
NB = KN // KB

_INF = float("inf")
_BIGI = 2**30


def _topk_body(x_ref, a_ref, w_ref, ix_ref, vals, idxs):
    k = pl.program_id(0)

    @pl.when(k == 0)
    def _init():
        vals[...] = jnp.full((16, QN), _INF, jnp.float32)
        idxs[...] = jnp.zeros((16, QN), jnp.int32)

    a = a_ref[...]
    x = x_ref[...]
    a2 = jnp.sum(a * a, axis=1, keepdims=True)
    dot = lax.dot_general(a, x, (((1,), (1,)), ((), ())),
                          preferred_element_type=jnp.float32)
    s = a2 - 2.0 * dot

    riota = lax.broadcasted_iota(jnp.int32, (KB, QN), 0)
    for j in range(TOP):
        m = jnp.min(s, axis=0, keepdims=True)
        p = jnp.min(jnp.where(s == m, riota, _BIGI), axis=0, keepdims=True)
        vals[pl.ds(TOP + j, 1), :] = m
        idxs[pl.ds(TOP + j, 1), :] = p + k * KB
        s = jnp.where(s == m, _INF, s)

    v = vals[...]
    ix = idxs[...]
    r16 = lax.broadcasted_iota(jnp.int32, (16, QN), 0)
    nv, ni = [], []
    for j in range(TOP):
        m = jnp.min(v, axis=0, keepdims=True)
        p = jnp.min(jnp.where(v == m, r16, _BIGI), axis=0, keepdims=True)
        sel = r16 == p
        ni.append(jnp.min(jnp.where(sel, ix, _BIGI), axis=0, keepdims=True))
        nv.append(m)
        v = jnp.where(sel, _INF, v)
    vals[...] = jnp.concatenate(nv + [jnp.full((16 - TOP, QN), _INF, jnp.float32)], axis=0)
    idxs[...] = jnp.concatenate(ni + [jnp.zeros((16 - TOP, QN), jnp.int32)], axis=0)

    @pl.when(k == NB - 1)
    def _fin():
        b2 = jnp.sum(x * x, axis=1)[None, :]
        vv = vals[pl.ds(0, 8), :]
        d2 = jnp.maximum(vv + b2, 0.0)
        w_ref[...] = 1.0 / jnp.sqrt(d2)
        ix_ref[...] = idxs[pl.ds(0, 8), :]


def _topk_stage(X, clusters):
    return pl.pallas_call(
        _topk_body,
        grid=(NB,),
        in_specs=[
            pl.BlockSpec((QN, DN), lambda k: (0, 0)),
            pl.BlockSpec((KB, DN), lambda k: (k, 0)),
        ],
        out_specs=[
            pl.BlockSpec((8, QN), lambda k: (0, 0)),
            pl.BlockSpec((8, QN), lambda k: (0, 0)),
        ],
        out_shape=[
            jax.ShapeDtypeStruct((8, QN), jnp.float32),
            jax.ShapeDtypeStruct((8, QN), jnp.int32),
        ],
        scratch_shapes=[
            pltpu.VMEM((16, QN), jnp.float32),
            pltpu.VMEM((16, QN), jnp.int32),
        ],
    )(X, clusters)


_NC = 2
_NS = 16
_NW = _NC * _NS
_QPW = QN // _NW
_NG = QN // 16


def _scatter_sc_body(w_hbm, ix_hbm, labels_hbm, counts_hbm, out_hbm,
                     labels_v, counts_v, w_v, i_v, rows_v):
    wid = lax.axis_index("s") * _NC + lax.axis_index("c")
    pltpu.sync_copy(labels_hbm, labels_v)
    pltpu.sync_copy(counts_hbm, counts_v.at[pl.ds(0, CN)])
    lane = lax.iota(jnp.int32, 16)
    for g in range(_QPW // 16):
        gidx = wid * (_QPW // 16) + g
        qb = gidx * 16
        pltpu.sync_copy(w_hbm.at[gidx], w_v)
        pltpu.sync_copy(ix_hbm.at[gidx], i_v)

        def _base(c, carry):
            off = jnp.minimum(c * 16, CN - 16)
            cv = counts_v[pl.ds(off, 16)]
            bv = jnp.where(cv == 0.0, jnp.float32(-1.0), jnp.float32(0.0))
            for r in range(16):
                rows_v[r, pl.ds(off, 16)] = bv
            return carry

        lax.fori_loop(0, (CN + 15) // 16, _base, 0)
        for j in range(TOP):
            lbl = plsc.load_gather(labels_v, [i_v[j, :]])
            cnt = plsc.load_gather(counts_v, [lbl])
            plsc.addupdate_scatter(rows_v, [lane, lbl], w_v[j, :] * cnt)
        pltpu.sync_copy(rows_v, out_hbm.at[pl.ds(qb, 16), :])


@functools.cache
def _scatter_sc():
    return functools.partial(
        pl.kernel,
        mesh=plsc.VectorSubcoreMesh(core_axis_name="c", subcore_axis_name="s"),
        out_type=jax.ShapeDtypeStruct((QN, CN), jnp.float32),
        scratch_types=[
            pltpu.VMEM((KN,), jnp.int32),
            pltpu.VMEM((CP,), jnp.float32),
            pltpu.VMEM((8, 16), jnp.float32),
            pltpu.VMEM((8, 16), jnp.int32),
            pltpu.VMEM((16, CN), jnp.float32),
        ],
        compiler_params=pltpu.CompilerParams(needs_layout_passes=False),
    )(_scatter_sc_body)


def kernel(X, clusters, counts, labels):
    wT, ixT = _topk_stage(X, clusters)
    w_grp = wT.reshape(8, _NG, 16).transpose(1, 0, 2)
    ix_grp = ixT.reshape(8, _NG, 16).transpose(1, 0, 2)
    return _scatter_sc()(w_grp, ix_grp, labels, counts)


# speedup vs baseline: 1.1975x; 1.0272x over previous
import functools

import jax
import jax.numpy as jnp
from jax import lax
from jax.experimental import pallas as pl
from jax.experimental.pallas import tpu as pltpu
from jax.experimental.pallas import tpu_sc as plsc

QN = 1024
KN = 16384
DN = 256
CN = 1000
CP = 1024
TOP = 5
KB = 2048
NB = KN // KB

_INF = float("inf")
_BIGI = 2**30


def _topk_body(x_ref, a_ref, w_ref, ix_ref, vals, idxs):
    k = pl.program_id(0)

    @pl.when(k == 0)
    def _init():
        vals[...] = jnp.full((16, QN), _INF, jnp.float32)
        idxs[...] = jnp.zeros((16, QN), jnp.int32)

    a = a_ref[...]
    x = x_ref[...]
    a2 = jnp.sum(a * a, axis=1, keepdims=True)
    dot = lax.dot_general(a, x, (((1,), (1,)), ((), ())),
                          preferred_element_type=jnp.float32)
    s = a2 - 2.0 * dot

    riota = lax.broadcasted_iota(jnp.int32, (KB, QN), 0)
    for j in range(TOP):
        m = jnp.min(s, axis=0, keepdims=True)
        p = jnp.min(jnp.where(s == m, riota, _BIGI), axis=0, keepdims=True)
        vals[pl.ds(TOP + j, 1), :] = m
        idxs[pl.ds(TOP + j, 1), :] = p + k * KB
        s = jnp.where(s == m, _INF, s)

    v = vals[...]
    ix = idxs[...]
    r16 = lax.broadcasted_iota(jnp.int32, (16, QN), 0)
    nv, ni = [], []
    for j in range(TOP):
        m = jnp.min(v, axis=0, keepdims=True)
        p = jnp.min(jnp.where(v == m, r16, _BIGI), axis=0, keepdims=True)
        sel = r16 == p
        ni.append(jnp.min(jnp.where(sel, ix, _BIGI), axis=0, keepdims=True))
        nv.append(m)
        v = jnp.where(sel, _INF, v)
    vals[...] = jnp.concatenate(nv + [jnp.full((16 - TOP, QN), _INF, jnp.float32)], axis=0)
    idxs[...] = jnp.concatenate(ni + [jnp.zeros((16 - TOP, QN), jnp.int32)], axis=0)

    @pl.when(k == NB - 1)
    def _fin():
        b2 = jnp.sum(x * x, axis=1)[None, :]
        vv = vals[pl.ds(0, 8), :]
        d2 = jnp.maximum(vv + b2, 0.0)
        w_ref[...] = 1.0 / jnp.sqrt(d2)
        ix_ref[...] = idxs[pl.ds(0, 8), :]


def _topk_stage(X, clusters):
    return pl.pallas_call(
        _topk_body,
        grid=(NB,),
        in_specs=[
            pl.BlockSpec((QN, DN), lambda k: (0, 0)),
            pl.BlockSpec((KB, DN), lambda k: (k, 0)),
        ],
        out_specs=[
            pl.BlockSpec((8, QN), lambda k: (0, 0)),
            pl.BlockSpec((8, QN), lambda k: (0, 0)),
        ],
        out_shape=[
            jax.ShapeDtypeStruct((8, QN), jnp.float32),
            jax.ShapeDtypeStruct((8, QN), jnp.int32),
        ],
        scratch_shapes=[
            pltpu.VMEM((16, QN), jnp.float32),
            pltpu.VMEM((16, QN), jnp.int32),
        ],
    )(X, clusters)


_NC = 2
_NS = 16
_NW = _NC * _NS
_QPW = QN // _NW
_NG = QN // 16


def _scatter_sc_body(w_hbm, ix_hbm, labels_hbm, counts_hbm, out_hbm,
                     labels_v, counts_v, w_v, i_v, rows_v):
    wid = lax.axis_index("s") * _NC + lax.axis_index("c")
    pltpu.sync_copy(labels_hbm, labels_v)
    pltpu.sync_copy(counts_hbm, counts_v.at[pl.ds(0, CN)])
    lane = lax.iota(jnp.int32, 16)
    for g in range(_QPW // 16):
        gidx = wid * (_QPW // 16) + g
        qb = gidx * 16
        pltpu.sync_copy(w_hbm.at[gidx], w_v)
        pltpu.sync_copy(ix_hbm.at[gidx], i_v)

        def _base(c, carry):
            off = jnp.minimum(c * 16, CN - 16)
            cv = counts_v[pl.ds(off, 16)]
            bv = jnp.where(cv == 0.0, jnp.float32(-1.0), jnp.float32(0.0))
            for r in range(16):
                rows_v[r, pl.ds(off, 16)] = bv
            return carry

        lax.fori_loop(0, (CN + 15) // 16, _base, 0)
        for j in range(TOP):
            lbl = plsc.load_gather(labels_v, [i_v[j, :]])
            cnt = plsc.load_gather(counts_v, [lbl])
            plsc.addupdate_scatter(rows_v, [lane, lbl], w_v[j, :] * cnt)
        pltpu.sync_copy(rows_v, out_hbm.at[pl.ds(qb, 16), :])


@functools.cache
def _scatter_sc():
    return functools.partial(
        pl.kernel,
        mesh=plsc.VectorSubcoreMesh(core_axis_name="c", subcore_axis_name="s"),
        out_type=jax.ShapeDtypeStruct((QN, CN), jnp.float32),
        scratch_types=[
            pltpu.VMEM((KN,), jnp.int32),
            pltpu.VMEM((CP,), jnp.float32),
            pltpu.VMEM((8, 16), jnp.float32),
            pltpu.VMEM((8, 16), jnp.int32),
            pltpu.VMEM((16, CN), jnp.float32),
        ],
        compiler_params=pltpu.CompilerParams(needs_layout_passes=False),
    )(_scatter_sc_body)


def kernel(X, clusters, counts, labels):
    wT, ixT = _topk_stage(X, clusters)
    w_grp = wT.reshape(8, _NG, 16).transpose(1, 0, 2)
    ix_grp = ixT.reshape(8, _NG, 16).transpose(1, 0, 2)
    return _scatter_sc()(w_grp, ix_grp, labels, counts)


# speedup vs baseline: 1.2253x; 1.0232x over previous
import functools

import jax
import jax.numpy as jnp
from jax import lax
from jax.experimental import pallas as pl
from jax.experimental.pallas import tpu as pltpu
from jax.experimental.pallas import tpu_sc as plsc

QN = 1024
KN = 16384
DN = 256
CN = 1000
CP = 1024
TOP = 5
KB = 4096
NB = KN // KB

_INF = float("inf")
_BIGI = 2**30


def _topk_body(x_ref, a_ref, w_ref, ix_ref, vals, idxs):
    k = pl.program_id(0)

    @pl.when(k == 0)
    def _init():
        vals[...] = jnp.full((16, QN), _INF, jnp.float32)
        idxs[...] = jnp.zeros((16, QN), jnp.int32)

    a = a_ref[...]
    x = x_ref[...]
    a2 = jnp.sum(a * a, axis=1, keepdims=True)
    dot = lax.dot_general(a, x, (((1,), (1,)), ((), ())),
                          preferred_element_type=jnp.float32)
    s = a2 - 2.0 * dot

    riota = lax.broadcasted_iota(jnp.int32, (KB, QN), 0)
    for j in range(TOP):
        m = jnp.min(s, axis=0, keepdims=True)
        p = jnp.min(jnp.where(s == m, riota, _BIGI), axis=0, keepdims=True)
        vals[pl.ds(TOP + j, 1), :] = m
        idxs[pl.ds(TOP + j, 1), :] = p + k * KB
        s = jnp.where(s == m, _INF, s)

    v = vals[...]
    ix = idxs[...]
    r16 = lax.broadcasted_iota(jnp.int32, (16, QN), 0)
    nv, ni = [], []
    for j in range(TOP):
        m = jnp.min(v, axis=0, keepdims=True)
        p = jnp.min(jnp.where(v == m, r16, _BIGI), axis=0, keepdims=True)
        sel = r16 == p
        ni.append(jnp.min(jnp.where(sel, ix, _BIGI), axis=0, keepdims=True))
        nv.append(m)
        v = jnp.where(sel, _INF, v)
    vals[...] = jnp.concatenate(nv + [jnp.full((16 - TOP, QN), _INF, jnp.float32)], axis=0)
    idxs[...] = jnp.concatenate(ni + [jnp.zeros((16 - TOP, QN), jnp.int32)], axis=0)

    @pl.when(k == NB - 1)
    def _fin():
        b2 = jnp.sum(x * x, axis=1)[None, :]
        vv = vals[pl.ds(0, 8), :]
        d2 = jnp.maximum(vv + b2, 0.0)
        w_ref[...] = 1.0 / jnp.sqrt(d2)
        ix_ref[...] = idxs[pl.ds(0, 8), :]


def _topk_stage(X, clusters):
    return pl.pallas_call(
        _topk_body,
        grid=(NB,),
        in_specs=[
            pl.BlockSpec((QN, DN), lambda k: (0, 0)),
            pl.BlockSpec((KB, DN), lambda k: (k, 0)),
        ],
        out_specs=[
            pl.BlockSpec((8, QN), lambda k: (0, 0)),
            pl.BlockSpec((8, QN), lambda k: (0, 0)),
        ],
        out_shape=[
            jax.ShapeDtypeStruct((8, QN), jnp.float32),
            jax.ShapeDtypeStruct((8, QN), jnp.int32),
        ],
        scratch_shapes=[
            pltpu.VMEM((16, QN), jnp.float32),
            pltpu.VMEM((16, QN), jnp.int32),
        ],
    )(X, clusters)


_NC = 2
_NS = 16
_NW = _NC * _NS
_QPW = QN // _NW
_NG = QN // 16


def _scatter_sc_body(w_hbm, ix_hbm, labels_hbm, counts_hbm, out_hbm,
                     labels_v, counts_v, w_v, i_v, rows_v):
    wid = lax.axis_index("s") * _NC + lax.axis_index("c")
    pltpu.sync_copy(labels_hbm, labels_v)
    pltpu.sync_copy(counts_hbm, counts_v.at[pl.ds(0, CN)])
    lane = lax.iota(jnp.int32, 16)
    for g in range(_QPW // 16):
        gidx = wid * (_QPW // 16) + g
        qb = gidx * 16
        pltpu.sync_copy(w_hbm.at[gidx], w_v)
        pltpu.sync_copy(ix_hbm.at[gidx], i_v)

        def _base(c, carry):
            off = jnp.minimum(c * 16, CN - 16)
            cv = counts_v[pl.ds(off, 16)]
            bv = jnp.where(cv == 0.0, jnp.float32(-1.0), jnp.float32(0.0))
            for r in range(16):
                rows_v[r, pl.ds(off, 16)] = bv
            return carry

        lax.fori_loop(0, (CN + 15) // 16, _base, 0)
        for j in range(TOP):
            lbl = plsc.load_gather(labels_v, [i_v[j, :]])
            cnt = plsc.load_gather(counts_v, [lbl])
            plsc.addupdate_scatter(rows_v, [lane, lbl], w_v[j, :] * cnt)
        pltpu.sync_copy(rows_v, out_hbm.at[pl.ds(qb, 16), :])


@functools.cache
def _scatter_sc():
    return functools.partial(
        pl.kernel,
        mesh=plsc.VectorSubcoreMesh(core_axis_name="c", subcore_axis_name="s"),
        out_type=jax.ShapeDtypeStruct((QN, CN), jnp.float32),
        scratch_types=[
            pltpu.VMEM((KN,), jnp.int32),
            pltpu.VMEM((CP,), jnp.float32),
            pltpu.VMEM((8, 16), jnp.float32),
            pltpu.VMEM((8, 16), jnp.int32),
            pltpu.VMEM((16, CN), jnp.float32),
        ],
        compiler_params=pltpu.CompilerParams(needs_layout_passes=False),
    )(_scatter_sc_body)


def kernel(X, clusters, counts, labels):
    wT, ixT = _topk_stage(X, clusters)
    w_grp = wT.reshape(8, _NG, 16).transpose(1, 0, 2)
    ix_grp = ixT.reshape(8, _NG, 16).transpose(1, 0, 2)
    return _scatter_sc()(w_grp, ix_grp, labels, counts)


# speedup vs baseline: 1.2477x; 1.0182x over previous
import functools

import jax
import jax.numpy as jnp
from jax import lax
from jax.experimental import pallas as pl
from jax.experimental.pallas import tpu as pltpu
from jax.experimental.pallas import tpu_sc as plsc

QN = 1024
KN = 16384
DN = 256
CN = 1000
CP = 1024
TOP = 5
KB = 4096
NB = KN // KB

_INF = float("inf")
_BIGI = 2**30


def _topk_body(x_ref, a_ref, w_ref, ix_ref, vals, idxs):
    k = pl.program_id(0)

    @pl.when(k == 0)
    def _init():
        vals[...] = jnp.full((16, QN), _INF, jnp.float32)
        idxs[...] = jnp.zeros((16, QN), jnp.int32)

    a = a_ref[...]
    x = x_ref[...]
    a2 = jnp.sum(a * a, axis=1, keepdims=True)
    dot = lax.dot_general(a, x, (((1,), (1,)), ((), ())),
                          preferred_element_type=jnp.float32)
    s = a2 - 2.0 * dot

    riota = lax.broadcasted_iota(jnp.int32, (KB, QN), 0)
    for j in range(TOP):
        m = jnp.min(s, axis=0, keepdims=True)
        p = jnp.min(jnp.where(s == m, riota, _BIGI), axis=0, keepdims=True)
        vals[pl.ds(TOP + j, 1), :] = m
        idxs[pl.ds(TOP + j, 1), :] = p + k * KB
        s = jnp.where(s == m, _INF, s)

    v = vals[...]
    ix = idxs[...]
    r16 = lax.broadcasted_iota(jnp.int32, (16, QN), 0)
    nv, ni = [], []
    for j in range(TOP):
        m = jnp.min(v, axis=0, keepdims=True)
        p = jnp.min(jnp.where(v == m, r16, _BIGI), axis=0, keepdims=True)
        sel = r16 == p
        ni.append(jnp.min(jnp.where(sel, ix, _BIGI), axis=0, keepdims=True))
        nv.append(m)
        v = jnp.where(sel, _INF, v)
    vals[...] = jnp.concatenate(nv + [jnp.full((16 - TOP, QN), _INF, jnp.float32)], axis=0)
    idxs[...] = jnp.concatenate(ni + [jnp.zeros((16 - TOP, QN), jnp.int32)], axis=0)

    @pl.when(k == NB - 1)
    def _fin():
        b2 = jnp.sum(x * x, axis=1)[None, :]
        vv = vals[pl.ds(0, 8), :]
        d2 = jnp.maximum(vv + b2, 0.0)
        w_ref[...] = 1.0 / jnp.sqrt(d2)
        ix_ref[...] = idxs[pl.ds(0, 8), :]


def _topk_stage(X, clusters):
    return pl.pallas_call(
        _topk_body,
        grid=(NB,),
        in_specs=[
            pl.BlockSpec((QN, DN), lambda k: (0, 0)),
            pl.BlockSpec((KB, DN), lambda k: (k, 0)),
        ],
        out_specs=[
            pl.BlockSpec((8, QN), lambda k: (0, 0)),
            pl.BlockSpec((8, QN), lambda k: (0, 0)),
        ],
        out_shape=[
            jax.ShapeDtypeStruct((8, QN), jnp.float32),
            jax.ShapeDtypeStruct((8, QN), jnp.int32),
        ],
        scratch_shapes=[
            pltpu.VMEM((16, QN), jnp.float32),
            pltpu.VMEM((16, QN), jnp.int32),
        ],
    )(X, clusters)


_NC = 2
_NS = 16
_NW = _NC * _NS
_QPW = QN // _NW
_NG = QN // 16


def _scatter_sc_body(w_hbm, ix_hbm, labels_hbm, counts_hbm, out_hbm,
                     labels_v, counts_v, w_v, i_v, rows_v):
    wid = lax.axis_index("s") * _NC + lax.axis_index("c")
    pltpu.sync_copy(labels_hbm, labels_v)
    pltpu.sync_copy(counts_hbm, counts_v.at[pl.ds(0, CN)])
    pltpu.sync_copy(w_hbm, w_v)
    pltpu.sync_copy(ix_hbm, i_v)
    lane = lax.iota(jnp.int32, 16)
    for g in range(_QPW // 16):
        gidx = wid * (_QPW // 16) + g
        qb = gidx * 16

        def _base(c, carry):
            off = jnp.minimum(c * 16, CN - 16)
            cv = counts_v[pl.ds(off, 16)]
            bv = jnp.where(cv == 0.0, jnp.float32(-1.0), jnp.float32(0.0))
            for r in range(16):
                rows_v[r, pl.ds(off, 16)] = bv
            return carry

        lax.fori_loop(0, (CN + 15) // 16, _base, 0)
        for j in range(TOP):
            lbl = plsc.load_gather(labels_v, [i_v[j, pl.ds(qb, 16)]])
            cnt = plsc.load_gather(counts_v, [lbl])
            plsc.addupdate_scatter(rows_v, [lane, lbl],
                                   w_v[j, pl.ds(qb, 16)] * cnt)
        pltpu.sync_copy(rows_v, out_hbm.at[pl.ds(qb, 16), :])


@functools.cache
def _scatter_sc():
    return functools.partial(
        pl.kernel,
        mesh=plsc.VectorSubcoreMesh(core_axis_name="c", subcore_axis_name="s"),
        out_type=jax.ShapeDtypeStruct((QN, CN), jnp.float32),
        scratch_types=[
            pltpu.VMEM((KN,), jnp.int32),
            pltpu.VMEM((CP,), jnp.float32),
            pltpu.VMEM((8, QN), jnp.float32),
            pltpu.VMEM((8, QN), jnp.int32),
            pltpu.VMEM((16, CN), jnp.float32),
        ],
        compiler_params=pltpu.CompilerParams(needs_layout_passes=False),
    )(_scatter_sc_body)


def kernel(X, clusters, counts, labels):
    wT, ixT = _topk_stage(X, clusters)
    return _scatter_sc()(wT, ixT, labels, counts)


# speedup vs baseline: 1.2499x; 1.0018x over previous
import functools

import jax
import jax.numpy as jnp
from jax import lax
from jax.experimental import pallas as pl
from jax.experimental.pallas import tpu as pltpu
from jax.experimental.pallas import tpu_sc as plsc

QN = 1024
KN = 16384
DN = 256
CN = 1000
CP = 1024
TOP = 5
KB = 4096
NB = KN // KB

_INF = float("inf")
_BIGI = 2**30


def _topk_body(x_ref, a_ref, w_ref, ix_ref, vals, idxs):
    k = pl.program_id(0)

    @pl.when(k == 0)
    def _init():
        vals[...] = jnp.full((16, QN), _INF, jnp.float32)
        idxs[...] = jnp.zeros((16, QN), jnp.int32)

    a = a_ref[...]
    x = x_ref[...]
    a2 = jnp.sum(a * a, axis=1, keepdims=True)
    dot = lax.dot_general(a, x, (((1,), (1,)), ((), ())),
                          preferred_element_type=jnp.float32)
    s = a2 - 2.0 * dot

    riota = lax.broadcasted_iota(jnp.int32, (KB, QN), 0)
    for j in range(TOP):
        m = jnp.min(s, axis=0, keepdims=True)
        p = jnp.min(jnp.where(s == m, riota, _BIGI), axis=0, keepdims=True)
        vals[pl.ds(TOP + j, 1), :] = m
        idxs[pl.ds(TOP + j, 1), :] = p + k * KB
        if j < TOP - 1:
            s = jnp.where(s == m, _INF, s)

    v = vals[...]
    ix = idxs[...]
    r16 = lax.broadcasted_iota(jnp.int32, (16, QN), 0)
    nv, ni = [], []
    for j in range(TOP):
        m = jnp.min(v, axis=0, keepdims=True)
        p = jnp.min(jnp.where(v == m, r16, _BIGI), axis=0, keepdims=True)
        sel = r16 == p
        ni.append(jnp.min(jnp.where(sel, ix, _BIGI), axis=0, keepdims=True))
        nv.append(m)
        v = jnp.where(sel, _INF, v)
    vals[...] = jnp.concatenate(nv + [jnp.full((16 - TOP, QN), _INF, jnp.float32)], axis=0)
    idxs[...] = jnp.concatenate(ni + [jnp.zeros((16 - TOP, QN), jnp.int32)], axis=0)

    @pl.when(k == NB - 1)
    def _fin():
        b2 = jnp.sum(x * x, axis=1)[None, :]
        vv = vals[pl.ds(0, 8), :]
        d2 = jnp.maximum(vv + b2, 0.0)
        w_ref[...] = 1.0 / jnp.sqrt(d2)
        ix_ref[...] = idxs[pl.ds(0, 8), :]


def _topk_stage(X, clusters):
    return pl.pallas_call(
        _topk_body,
        grid=(NB,),
        in_specs=[
            pl.BlockSpec((QN, DN), lambda k: (0, 0)),
            pl.BlockSpec((KB, DN), lambda k: (k, 0)),
        ],
        out_specs=[
            pl.BlockSpec((8, QN), lambda k: (0, 0)),
            pl.BlockSpec((8, QN), lambda k: (0, 0)),
        ],
        out_shape=[
            jax.ShapeDtypeStruct((8, QN), jnp.float32),
            jax.ShapeDtypeStruct((8, QN), jnp.int32),
        ],
        scratch_shapes=[
            pltpu.VMEM((16, QN), jnp.float32),
            pltpu.VMEM((16, QN), jnp.int32),
        ],
    )(X, clusters)


_NC = 2
_NS = 16
_NW = _NC * _NS
_QPW = QN // _NW
_NG = QN // 16


def _scatter_sc_body(w_hbm, ix_hbm, labels_hbm, counts_hbm, out_hbm,
                     labels_v, counts_v, w_v, i_v, rows_v):
    wid = lax.axis_index("s") * _NC + lax.axis_index("c")
    pltpu.sync_copy(labels_hbm, labels_v)
    pltpu.sync_copy(counts_hbm, counts_v.at[pl.ds(0, CN)])
    pltpu.sync_copy(w_hbm, w_v)
    pltpu.sync_copy(ix_hbm, i_v)
    lane = lax.iota(jnp.int32, 16)
    for g in range(_QPW // 16):
        gidx = wid * (_QPW // 16) + g
        qb = gidx * 16

        def _base(c, carry):
            off = jnp.minimum(c * 16, CN - 16)
            cv = counts_v[pl.ds(off, 16)]
            bv = jnp.where(cv == 0.0, jnp.float32(-1.0), jnp.float32(0.0))
            for r in range(16):
                rows_v[r, pl.ds(off, 16)] = bv
            return carry

        lax.fori_loop(0, (CN + 15) // 16, _base, 0)
        for j in range(TOP):
            lbl = plsc.load_gather(labels_v, [i_v[j, pl.ds(qb, 16)]])
            cnt = plsc.load_gather(counts_v, [lbl])
            plsc.addupdate_scatter(rows_v, [lane, lbl],
                                   w_v[j, pl.ds(qb, 16)] * cnt)
        pltpu.sync_copy(rows_v, out_hbm.at[pl.ds(qb, 16), :])


@functools.cache
def _scatter_sc():
    return functools.partial(
        pl.kernel,
        mesh=plsc.VectorSubcoreMesh(core_axis_name="c", subcore_axis_name="s"),
        out_type=jax.ShapeDtypeStruct((QN, CN), jnp.float32),
        scratch_types=[
            pltpu.VMEM((KN,), jnp.int32),
            pltpu.VMEM((CP,), jnp.float32),
            pltpu.VMEM((8, QN), jnp.float32),
            pltpu.VMEM((8, QN), jnp.int32),
            pltpu.VMEM((16, CN), jnp.float32),
        ],
        compiler_params=pltpu.CompilerParams(needs_layout_passes=False),
    )(_scatter_sc_body)


def kernel(X, clusters, counts, labels):
    wT, ixT = _topk_stage(X, clusters)
    return _scatter_sc()(wT, ixT, labels, counts)
